# baseline (speedup 1.0000x reference)
import functools

import jax
import jax.numpy as jnp
from jax import lax
from jax.experimental import pallas as pl
from jax.experimental.pallas import tpu as pltpu
from jax.experimental.pallas import tpu_sc as plsc

N_USERS_K = 50000
N_ITEMS_K = 50000
NN = N_USERS_K + N_ITEMS_K
DD = 32
EE = 1600000
BB = 4096
ALPHA = 0.5
LAYERS = 3

NC = 2
NS = 16
LANES = 16

CH = 128
NCHUNK = 784
E_PAD = NS * CH * NCHUNK
EPT = E_PAD // NS
NN_PAD = 100096
ROWS_PT = NN_PAD // NS
ZR = 784

_mesh = plsc.VectorSubcoreMesh(core_axis_name="c", subcore_axis_name="s")


def _spmv_body(emb2, rows, cols, vals, out0, out1,
               colb, rowb, valb, gb, zb, acc, sem_st, sem_g0, sem_g1):
    c = lax.axis_index("c")
    s = lax.axis_index("s")
    sem_g = (sem_g0, sem_g1)

    def zbody(i, carry):
        zb[i, :] = jnp.zeros((LANES,), jnp.float32)
        return carry
    lax.fori_loop(0, ZR, zbody, 0, unroll=8)
    for k in range(7):
        pltpu.sync_copy(zb, acc.at[pl.ds(s * ROWS_PT + k * ZR, ZR)])
    pltpu.sync_copy(zb.at[pl.ds(0, ROWS_PT - 7 * ZR)],
                    acc.at[pl.ds(s * ROWS_PT + 7 * ZR, ROWS_PT - 7 * ZR)])
    plsc.subcore_barrier()

    def issue_stage(j, b):
        base = s * EPT + j * CH
        pltpu.async_copy(rows.at[pl.ds(base, CH)], rowb.at[b], sem_st)
        pltpu.async_copy(cols.at[pl.ds(base, CH)], colb.at[b], sem_st)
        pltpu.async_copy(vals.at[pl.ds(base, CH)], valb.at[b], sem_st)

    def wait_stage(b):
        pltpu.make_async_copy(rows.at[pl.ds(0, CH)], rowb.at[b], sem_st).wait()
        pltpu.make_async_copy(cols.at[pl.ds(0, CH)], colb.at[b], sem_st).wait()
        pltpu.make_async_copy(vals.at[pl.ds(0, CH)], valb.at[b], sem_st).wait()

    def issue_gather(b):
        for i in range(CH // LANES):
            cv = colb[b, pl.ds(i * LANES, LANES)]
            colb[b, pl.ds(i * LANES, LANES)] = cv * 2 + c
        pltpu.async_copy(emb2.at[colb.at[b]], gb.at[b], sem_g[b])

    def wait_gather(b):
        pltpu.make_async_copy(emb2.at[colb.at[b]], gb.at[b], sem_g[b]).wait()

    def multiply(b):
        for i in range(CH // LANES):
            vv = valb[b, pl.ds(i * LANES, LANES)]
            for t in range(LANES):
                e = i * LANES + t
                gb[b, e, :] = gb[b, e, :] * vv[t]

    def scatter(b):
        pltpu.sync_copy(gb.at[b], acc.at[rowb.at[b]], add=True)

    issue_stage(0, 0)
    wait_stage(0)
    issue_gather(0)
    issue_stage(1, 1)

    def pair(g, carry):
        for b in range(2):
            j = 2 * g + b
            nb = 1 - b
            wait_stage(nb)
            issue_gather(nb)
            wait_gather(b)
            multiply(b)
            scatter(b)
            issue_stage(j + 2, b)
        return carry

    lax.fori_loop(0, (NCHUNK - 2) // 2, pair, 0)

    wait_stage(1)
    issue_gather(1)
    wait_gather(0)
    multiply(0)
    scatter(0)
    wait_gather(1)
    multiply(1)
    scatter(1)
    plsc.subcore_barrier()

    @pl.when(c == 0)
    def _():
        pltpu.sync_copy(acc.at[pl.ds(s * ROWS_PT, ROWS_PT)],
                        out0.at[pl.ds(s * ROWS_PT, ROWS_PT)])

    @pl.when(c == 1)
    def _():
        pltpu.sync_copy(acc.at[pl.ds(s * ROWS_PT, ROWS_PT)],
                        out1.at[pl.ds(s * ROWS_PT, ROWS_PT)])


_spmv = pl.kernel(
    _spmv_body,
    out_type=(jax.ShapeDtypeStruct((NN_PAD, 16), jnp.float32),
              jax.ShapeDtypeStruct((NN_PAD, 16), jnp.float32)),
    mesh=_mesh,
    scratch_types=[
        pltpu.VMEM((2, CH), jnp.int32),
        pltpu.VMEM((2, CH), jnp.int32),
        pltpu.VMEM((2, CH), jnp.float32),
        pltpu.VMEM((2, CH, 16), jnp.float32),
        pltpu.VMEM((ZR, 16), jnp.float32),
        pltpu.VMEM_SHARED((NN_PAD, 16), jnp.float32),
        pltpu.SemaphoreType.DMA,
        pltpu.SemaphoreType.DMA,
        pltpu.SemaphoreType.DMA,
    ],
    compiler_params=pltpu.CompilerParams(use_tc_tiling_on_sc=False),
    name="lgcn_spmv_sc",
)


def _blend_body(final_layer, old_ref, n0_ref, n1_ref, acc_ref,
                emb_out_ref, acc_out_ref):
    old = old_ref[...]
    new = jnp.concatenate([n0_ref[...], n1_ref[...]], axis=-1)
    diff = old - new + 1e-6
    os_score = jnp.sqrt(jnp.sum(diff * diff, axis=1, keepdims=True))
    d_new = ALPHA * jnp.log1p(os_score)
    inv = 1.0 / (1.0 + d_new)
    emb = (old + d_new * new) * inv
    emb_out_ref[...] = emb
    acc = acc_ref[...] + emb
    if final_layer:
        acc = acc * 0.25
    acc_out_ref[...] = acc


def _make_blend(final_layer):
    blk = 1000
    grid = NN // blk
    return pl.pallas_call(
        functools.partial(_blend_body, final_layer),
        grid=(grid,),
        in_specs=[
            pl.BlockSpec((blk, DD), lambda i: (i, 0)),
            pl.BlockSpec((blk, 16), lambda i: (i, 0)),
            pl.BlockSpec((blk, 16), lambda i: (i, 0)),
            pl.BlockSpec((blk, DD), lambda i: (i, 0)),
        ],
        out_specs=[
            pl.BlockSpec((blk, DD), lambda i: (i, 0)),
            pl.BlockSpec((blk, DD), lambda i: (i, 0)),
        ],
        out_shape=[
            jax.ShapeDtypeStruct((NN, DD), jnp.float32),
            jax.ShapeDtypeStruct((NN, DD), jnp.float32),
        ],
        name="lgcn_blend_tc",
    )


_blend_mid = _make_blend(False)
_blend_last = _make_blend(True)


IDS_PT = BB // (NC * NS)


def _take_body(final_hbm, uid, iid, out_u, out_i, idxb, rbuf, gsem):
    c = lax.axis_index("c")
    s = lax.axis_index("s")
    w = s * NC + c
    base = w * IDS_PT

    pltpu.sync_copy(uid.at[pl.ds(base, IDS_PT)], idxb.at[0])
    pltpu.async_copy(final_hbm.at[idxb.at[0]], rbuf, gsem).wait()
    pltpu.sync_copy(rbuf, out_u.at[pl.ds(base, IDS_PT)])

    pltpu.sync_copy(iid.at[pl.ds(base, IDS_PT)], idxb.at[0])
    for i in range(IDS_PT // LANES):
        iv = idxb[0, pl.ds(i * LANES, LANES)]
        idxb[0, pl.ds(i * LANES, LANES)] = iv + N_USERS_K
    pltpu.async_copy(final_hbm.at[idxb.at[0]], rbuf, gsem).wait()
    pltpu.sync_copy(rbuf, out_i.at[pl.ds(base, IDS_PT)])


_take = pl.kernel(
    _take_body,
    out_type=(jax.ShapeDtypeStruct((BB, DD), jnp.float32),
              jax.ShapeDtypeStruct((BB, DD), jnp.float32)),
    mesh=_mesh,
    scratch_types=[
        pltpu.VMEM((1, IDS_PT), jnp.int32),
        pltpu.VMEM((IDS_PT, DD), jnp.float32),
        pltpu.SemaphoreType.DMA,
    ],
    compiler_params=pltpu.CompilerParams(use_tc_tiling_on_sc=False),
    name="lgcn_take_sc",
)


def kernel(user_id, item_id, user_table, item_table, adj_row, adj_col, adj_vals):
    ego = jnp.concatenate([user_table, item_table], axis=0)
    pad = E_PAD - EE
    rows_p = jnp.concatenate([adj_row, jnp.zeros((pad,), jnp.int32)])
    cols_p = jnp.concatenate([adj_col, jnp.zeros((pad,), jnp.int32)])
    vals_p = jnp.concatenate([adj_vals, jnp.zeros((pad,), jnp.float32)])
    emb = ego
    acc = ego
    for layer in range(LAYERS):
        n0, n1 = _spmv(emb.reshape(2 * NN, 16), rows_p, cols_p, vals_p)
        blend = _blend_last if layer == LAYERS - 1 else _blend_mid
        emb, acc = blend(emb, n0, n1, acc)
    u_embed, i_embed = _take(acc, user_id, item_id)
    return (u_embed, i_embed)


# speedup vs baseline: 1.0131x; 1.0131x over previous
import functools

import jax
import jax.numpy as jnp
from jax import lax
from jax.experimental import pallas as pl
from jax.experimental.pallas import tpu as pltpu
from jax.experimental.pallas import tpu_sc as plsc

N_USERS_K = 50000
N_ITEMS_K = 50000
NN = N_USERS_K + N_ITEMS_K
DD = 32
EE = 1600000
BB = 4096
ALPHA = 0.5
LAYERS = 3

NC = 2
NS = 16
LANES = 16

CH = 128
NCHUNK = 784
E_PAD = NS * CH * NCHUNK
EPT = E_PAD // NS
NN_PAD = 100096
ROWS_PT = NN_PAD // NS
ZR = 784

_mesh = plsc.VectorSubcoreMesh(core_axis_name="c", subcore_axis_name="s")


def _spmv_body(elo, ehi, rows, cols, vals, out0, out1,
               colb, rowb, valb, gb, zb, acc,
               sem_st, sem_g0, sem_g1, sem_g2, sem_s0, sem_s1, sem_s2):
    c = lax.axis_index("c")
    s = lax.axis_index("s")
    sem_g = (sem_g0, sem_g1, sem_g2)
    sem_sc = (sem_s0, sem_s1, sem_s2)

    def zbody(i, carry):
        zb[i, :] = jnp.zeros((LANES,), jnp.float32)
        return carry
    lax.fori_loop(0, ZR, zbody, 0, unroll=8)
    for k in range(7):
        pltpu.sync_copy(zb, acc.at[pl.ds(s * ROWS_PT + k * ZR, ZR)])
    pltpu.sync_copy(zb.at[pl.ds(0, ROWS_PT - 7 * ZR)],
                    acc.at[pl.ds(s * ROWS_PT + 7 * ZR, ROWS_PT - 7 * ZR)])
    plsc.subcore_barrier()

    def issue_stage(j, b):
        base = s * EPT + j * CH
        pltpu.async_copy(rows.at[pl.ds(base, CH)], rowb.at[b], sem_st)
        pltpu.async_copy(cols.at[pl.ds(base, CH)], colb.at[b], sem_st)
        pltpu.async_copy(vals.at[pl.ds(base, CH)], valb.at[b], sem_st)

    def wait_stage(b):
        pltpu.make_async_copy(rows.at[pl.ds(0, CH)], rowb.at[b], sem_st).wait()
        pltpu.make_async_copy(cols.at[pl.ds(0, CH)], colb.at[b], sem_st).wait()
        pltpu.make_async_copy(vals.at[pl.ds(0, CH)], valb.at[b], sem_st).wait()

    def issue_gather(b):
        @pl.when(c == 0)
        def _():
            pltpu.async_copy(elo.at[colb.at[b]], gb.at[b], sem_g[b])

        @pl.when(c == 1)
        def _():
            pltpu.async_copy(ehi.at[colb.at[b]], gb.at[b], sem_g[b])

    def wait_gather(b):
        pltpu.make_async_copy(elo.at[colb.at[b]], gb.at[b], sem_g[b]).wait()

    def multiply(b):
        for i in range(CH // LANES):
            vv = valb[b, pl.ds(i * LANES, LANES)]
            for t in range(LANES):
                e = i * LANES + t
                gb[b, e, :] = gb[b, e, :] * vv[t]

    def issue_scatter(b):
        pltpu.async_copy(gb.at[b], acc.at[rowb.at[b]], sem_sc[b], add=True)

    def wait_scatter(b):
        pltpu.make_async_copy(gb.at[b], acc.at[rowb.at[b]], sem_sc[b]).wait()

    issue_stage(0, 0)
    wait_stage(0)
    issue_gather(0)
    issue_stage(1, 1)

    wait_stage(1)
    issue_gather(1)
    wait_gather(0)
    multiply(0)
    issue_scatter(0)
    issue_stage(2, 2)

    wait_stage(2)
    issue_gather(2)
    wait_gather(1)
    multiply(1)
    issue_scatter(1)
    wait_scatter(0)
    issue_stage(3, 0)

    def step(j, b):
        sn = (b + 1) % 3
        ss = (b + 2) % 3
        wait_stage(sn)
        issue_gather(sn)
        wait_gather(b)
        multiply(b)
        issue_scatter(b)
        wait_scatter(ss)
        issue_stage(j + 2, ss)

    def group(g, carry):
        j = 3 * g + 2
        step(j, 2)
        step(j + 1, 0)
        step(j + 2, 1)
        return carry

    lax.fori_loop(0, (NCHUNK - 4) // 3, group, 0)

    wait_stage(0)
    issue_gather(0)
    wait_gather(2)
    multiply(2)
    issue_scatter(2)
    wait_scatter(1)

    wait_gather(0)
    multiply(0)
    issue_scatter(0)
    wait_scatter(2)
    wait_scatter(0)
    plsc.subcore_barrier()

    @pl.when(c == 0)
    def _():
        pltpu.sync_copy(acc.at[pl.ds(s * ROWS_PT, ROWS_PT)],
                        out0.at[pl.ds(s * ROWS_PT, ROWS_PT)])

    @pl.when(c == 1)
    def _():
        pltpu.sync_copy(acc.at[pl.ds(s * ROWS_PT, ROWS_PT)],
                        out1.at[pl.ds(s * ROWS_PT, ROWS_PT)])


_spmv = pl.kernel(
    _spmv_body,
    out_type=(jax.ShapeDtypeStruct((NN_PAD, 16), jnp.float32),
              jax.ShapeDtypeStruct((NN_PAD, 16), jnp.float32)),
    mesh=_mesh,
    scratch_types=[
        pltpu.VMEM((3, CH), jnp.int32),
        pltpu.VMEM((3, CH), jnp.int32),
        pltpu.VMEM((3, CH), jnp.float32),
        pltpu.VMEM((3, CH, 16), jnp.float32),
        pltpu.VMEM((ZR, 16), jnp.float32),
        pltpu.VMEM_SHARED((NN_PAD, 16), jnp.float32),
        pltpu.SemaphoreType.DMA,
        pltpu.SemaphoreType.DMA,
        pltpu.SemaphoreType.DMA,
        pltpu.SemaphoreType.DMA,
        pltpu.SemaphoreType.DMA,
        pltpu.SemaphoreType.DMA,
        pltpu.SemaphoreType.DMA,
    ],
    compiler_params=pltpu.CompilerParams(use_tc_tiling_on_sc=False),
    name="lgcn_spmv_sc",
)


def _blend_body(final_layer, old_ref, n0_ref, n1_ref, acc_ref,
                emb_out_ref, acc_out_ref, lo_out_ref, hi_out_ref):
    old = old_ref[...]
    new = jnp.concatenate([n0_ref[...], n1_ref[...]], axis=-1)
    diff = old - new + 1e-6
    os_score = jnp.sqrt(jnp.sum(diff * diff, axis=1, keepdims=True))
    d_new = ALPHA * jnp.log1p(os_score)
    inv = 1.0 / (1.0 + d_new)
    emb = (old + d_new * new) * inv
    emb_out_ref[...] = emb
    lo_out_ref[...] = emb[:, :16]
    hi_out_ref[...] = emb[:, 16:]
    acc = acc_ref[...] + emb
    if final_layer:
        acc = acc * 0.25
    acc_out_ref[...] = acc


def _make_blend(final_layer):
    blk = 6256
    grid = NN_PAD // blk
    return pl.pallas_call(
        functools.partial(_blend_body, final_layer),
        grid=(grid,),
        in_specs=[
            pl.BlockSpec((blk, DD), lambda i: (i, 0)),
            pl.BlockSpec((blk, 16), lambda i: (i, 0)),
            pl.BlockSpec((blk, 16), lambda i: (i, 0)),
            pl.BlockSpec((blk, DD), lambda i: (i, 0)),
        ],
        out_specs=[
            pl.BlockSpec((blk, DD), lambda i: (i, 0)),
            pl.BlockSpec((blk, DD), lambda i: (i, 0)),
            pl.BlockSpec((blk, 16), lambda i: (i, 0)),
            pl.BlockSpec((blk, 16), lambda i: (i, 0)),
        ],
        out_shape=[
            jax.ShapeDtypeStruct((NN_PAD, DD), jnp.float32),
            jax.ShapeDtypeStruct((NN_PAD, DD), jnp.float32),
            jax.ShapeDtypeStruct((NN_PAD, 16), jnp.float32),
            jax.ShapeDtypeStruct((NN_PAD, 16), jnp.float32),
        ],
        name="lgcn_blend_tc",
    )


_blend_mid = _make_blend(False)
_blend_last = _make_blend(True)


IDS_PT = BB // (NC * NS)


def _take_body(final_hbm, uid, iid, out_u, out_i, idxb, rbuf, gsem):
    c = lax.axis_index("c")
    s = lax.axis_index("s")
    w = s * NC + c
    base = w * IDS_PT

    pltpu.sync_copy(uid.at[pl.ds(base, IDS_PT)], idxb.at[0])
    pltpu.async_copy(final_hbm.at[idxb.at[0]], rbuf, gsem).wait()
    pltpu.sync_copy(rbuf, out_u.at[pl.ds(base, IDS_PT)])

    pltpu.sync_copy(iid.at[pl.ds(base, IDS_PT)], idxb.at[0])
    for i in range(IDS_PT // LANES):
        iv = idxb[0, pl.ds(i * LANES, LANES)]
        idxb[0, pl.ds(i * LANES, LANES)] = iv + N_USERS_K
    pltpu.async_copy(final_hbm.at[idxb.at[0]], rbuf, gsem).wait()
    pltpu.sync_copy(rbuf, out_i.at[pl.ds(base, IDS_PT)])


_take = pl.kernel(
    _take_body,
    out_type=(jax.ShapeDtypeStruct((BB, DD), jnp.float32),
              jax.ShapeDtypeStruct((BB, DD), jnp.float32)),
    mesh=_mesh,
    scratch_types=[
        pltpu.VMEM((1, IDS_PT), jnp.int32),
        pltpu.VMEM((IDS_PT, DD), jnp.float32),
        pltpu.SemaphoreType.DMA,
    ],
    compiler_params=pltpu.CompilerParams(use_tc_tiling_on_sc=False),
    name="lgcn_take_sc",
)


def kernel(user_id, item_id, user_table, item_table, adj_row, adj_col, adj_vals):
    ego = jnp.concatenate(
        [user_table, item_table, jnp.zeros((NN_PAD - NN, DD), jnp.float32)],
        axis=0)
    pad = E_PAD - EE
    rows_p = jnp.concatenate([adj_row, jnp.zeros((pad,), jnp.int32)])
    cols_p = jnp.concatenate([adj_col, jnp.zeros((pad,), jnp.int32)])
    vals_p = jnp.concatenate([adj_vals, jnp.zeros((pad,), jnp.float32)])
    emb = ego
    acc = ego
    lo = ego[:, :16]
    hi = ego[:, 16:]
    for layer in range(LAYERS):
        n0, n1 = _spmv(lo, hi, rows_p, cols_p, vals_p)
        blend = _blend_last if layer == LAYERS - 1 else _blend_mid
        emb, acc, lo, hi = blend(emb, n0, n1, acc)
    u_embed, i_embed = _take(acc, user_id, item_id)
    return (u_embed, i_embed)


# speedup vs baseline: 1.3584x; 1.3408x over previous
import functools

import jax
import jax.numpy as jnp
from jax import lax
from jax.experimental import pallas as pl
from jax.experimental.pallas import tpu as pltpu
from jax.experimental.pallas import tpu_sc as plsc

N_USERS_K = 50000
N_ITEMS_K = 50000
NN = N_USERS_K + N_ITEMS_K
DD = 32
EE = 1600000
BB = 4096
ALPHA = 0.5
LAYERS = 3

NC = 2
NS = 16
LANES = 16

CH = 128
NCHUNK = 784
E_PAD = NS * CH * NCHUNK
EPT = E_PAD // NS
NN_PAD = 100096
ROWS_PT = NN_PAD // NS
ZR = 784

_mesh = plsc.VectorSubcoreMesh(core_axis_name="c", subcore_axis_name="s")


def _spmv_body(elo, ehi, rows, cols, vals, out0, out1,
               colb, rowb, valb, gb, zb, acc,
               sem_st, sem_g0, sem_g1, sem_g2, sem_s0, sem_s1, sem_s2):
    c = lax.axis_index("c")
    s = lax.axis_index("s")
    sem_g = (sem_g0, sem_g1, sem_g2)
    sem_sc = (sem_s0, sem_s1, sem_s2)

    def zbody(i, carry):
        zb[i, :] = jnp.zeros((LANES,), jnp.float32)
        return carry
    lax.fori_loop(0, ZR, zbody, 0, unroll=8)
    for k in range(7):
        pltpu.sync_copy(zb, acc.at[pl.ds(s * ROWS_PT + k * ZR, ZR)])
    pltpu.sync_copy(zb.at[pl.ds(0, ROWS_PT - 7 * ZR)],
                    acc.at[pl.ds(s * ROWS_PT + 7 * ZR, ROWS_PT - 7 * ZR)])
    plsc.subcore_barrier()

    def issue_stage(j, b):
        base = s * EPT + j * CH
        pltpu.async_copy(rows.at[pl.ds(base, CH)], rowb.at[b], sem_st)
        pltpu.async_copy(cols.at[pl.ds(base, CH)], colb.at[b], sem_st)
        pltpu.async_copy(vals.at[pl.ds(base, CH)], valb.at[b], sem_st)

    def wait_stage(b):
        pltpu.make_async_copy(rows.at[pl.ds(0, CH)], rowb.at[b], sem_st).wait()
        pltpu.make_async_copy(cols.at[pl.ds(0, CH)], colb.at[b], sem_st).wait()
        pltpu.make_async_copy(vals.at[pl.ds(0, CH)], valb.at[b], sem_st).wait()

    def issue_gather(b):
        @pl.when(c == 0)
        def _():
            pltpu.async_copy(elo.at[colb.at[b]], gb.at[b], sem_g[b])

        @pl.when(c == 1)
        def _():
            pltpu.async_copy(ehi.at[colb.at[b]], gb.at[b], sem_g[b])

    def wait_gather(b):
        pltpu.make_async_copy(elo.at[colb.at[b]], gb.at[b], sem_g[b]).wait()

    def multiply(b):
        for i in range(CH // LANES):
            vv = valb[b, pl.ds(i * LANES, LANES)]
            for t in range(LANES):
                e = i * LANES + t
                gb[b, e, :] = gb[b, e, :] * vv[t]

    def issue_scatter(b):
        pltpu.async_copy(gb.at[b], acc.at[rowb.at[b]], sem_sc[b], add=True)

    def wait_scatter(b):
        pltpu.make_async_copy(gb.at[b], acc.at[rowb.at[b]], sem_sc[b]).wait()

    issue_stage(0, 0)
    wait_stage(0)
    issue_gather(0)
    issue_stage(1, 1)

    wait_stage(1)
    issue_gather(1)
    wait_gather(0)
    multiply(0)
    issue_scatter(0)
    issue_stage(2, 2)

    wait_stage(2)
    issue_gather(2)
    wait_gather(1)
    multiply(1)
    issue_scatter(1)
    wait_scatter(0)
    issue_stage(3, 0)

    def step(j, b):
        sn = (b + 1) % 3
        ss = (b + 2) % 3
        wait_stage(sn)
        issue_gather(sn)
        wait_gather(b)
        multiply(b)
        issue_scatter(b)
        wait_scatter(ss)
        issue_stage(j + 2, ss)

    def group(g, carry):
        j = 3 * g + 2
        step(j, 2)
        step(j + 1, 0)
        step(j + 2, 1)
        return carry

    lax.fori_loop(0, (NCHUNK - 4) // 3, group, 0)

    wait_stage(0)
    issue_gather(0)
    wait_gather(2)
    multiply(2)
    issue_scatter(2)
    wait_scatter(1)

    wait_gather(0)
    multiply(0)
    issue_scatter(0)
    wait_scatter(2)
    wait_scatter(0)
    plsc.subcore_barrier()

    @pl.when(c == 0)
    def _():
        pltpu.sync_copy(acc.at[pl.ds(s * ROWS_PT, ROWS_PT)],
                        out0.at[pl.ds(s * ROWS_PT, ROWS_PT)])

    @pl.when(c == 1)
    def _():
        pltpu.sync_copy(acc.at[pl.ds(s * ROWS_PT, ROWS_PT)],
                        out1.at[pl.ds(s * ROWS_PT, ROWS_PT)])


_spmv = pl.kernel(
    _spmv_body,
    out_type=(jax.ShapeDtypeStruct((NN_PAD, 16), jnp.float32),
              jax.ShapeDtypeStruct((NN_PAD, 16), jnp.float32)),
    mesh=_mesh,
    scratch_types=[
        pltpu.VMEM((3, CH), jnp.int32),
        pltpu.VMEM((3, CH), jnp.int32),
        pltpu.VMEM((3, CH), jnp.float32),
        pltpu.VMEM((3, CH, 16), jnp.float32),
        pltpu.VMEM((ZR, 16), jnp.float32),
        pltpu.VMEM_SHARED((NN_PAD, 16), jnp.float32),
        pltpu.SemaphoreType.DMA,
        pltpu.SemaphoreType.DMA,
        pltpu.SemaphoreType.DMA,
        pltpu.SemaphoreType.DMA,
        pltpu.SemaphoreType.DMA,
        pltpu.SemaphoreType.DMA,
        pltpu.SemaphoreType.DMA,
    ],
    compiler_params=pltpu.CompilerParams(use_tc_tiling_on_sc=False),
    name="lgcn_spmv_sc",
)


NW = NN_PAD * 16 // 128


def _blend_body(final_layer, lo_ref, hi_ref, n0_ref, n1_ref,
                alo_ref, ahi_ref,
                lo_out, hi_out, alo_out, ahi_out):
    lo = lo_ref[...]
    hi = hi_ref[...]
    n0 = n0_ref[...]
    n1 = n1_ref[...]
    d0 = lo - n0 + 1e-6
    d1 = hi - n1 + 1e-6
    sq = d0 * d0 + d1 * d1
    lane = lax.broadcasted_iota(jnp.int32, (128, 8), 0)
    grp = lax.broadcasted_iota(jnp.int32, (128, 8), 1)
    seg = jnp.where(lane // 16 == grp, 1.0, 0.0).astype(jnp.float32)
    ssum = lax.dot_general(sq, seg, (((1,), (0,)), ((), ())),
                           precision=lax.Precision.HIGHEST,
                           preferred_element_type=jnp.float32)
    d_new = ALPHA * jnp.log1p(jnp.sqrt(ssum))
    inv = 1.0 / (1.0 + d_new)
    segT = seg.T
    dl = lax.dot_general(d_new, segT, (((1,), (0,)), ((), ())),
                         precision=lax.Precision.HIGHEST,
                         preferred_element_type=jnp.float32)
    il = lax.dot_general(inv, segT, (((1,), (0,)), ((), ())),
                         precision=lax.Precision.HIGHEST,
                         preferred_element_type=jnp.float32)
    emb_lo = (lo + dl * n0) * il
    emb_hi = (hi + dl * n1) * il
    lo_out[...] = emb_lo
    hi_out[...] = emb_hi
    alo = alo_ref[...] + emb_lo
    ahi = ahi_ref[...] + emb_hi
    if final_layer:
        alo = alo * 0.25
        ahi = ahi * 0.25
    alo_out[...] = alo
    ahi_out[...] = ahi


def _make_blend(final_layer):
    blk = 3128
    grid = NW // blk
    spec = pl.BlockSpec((blk, 128), lambda i: (i, 0))
    return pl.pallas_call(
        functools.partial(_blend_body, final_layer),
        grid=(grid,),
        in_specs=[spec] * 6,
        out_specs=[spec] * 4,
        out_shape=[jax.ShapeDtypeStruct((NW, 128), jnp.float32)] * 4,
        name="lgcn_blend_tc",
    )


_blend_mid = _make_blend(False)
_blend_last = _make_blend(True)


IDS2 = BB // NS


def _take_body(alo, ahi, uid, iid, out_u, out_i, idxb, rbu, rbi, gsem):
    c = lax.axis_index("c")
    s = lax.axis_index("s")
    base = s * IDS2

    pltpu.sync_copy(uid.at[pl.ds(base, IDS2)], idxb.at[0])
    pltpu.sync_copy(iid.at[pl.ds(base, IDS2)], idxb.at[1])
    for i in range(IDS2 // LANES):
        iv = idxb[1, pl.ds(i * LANES, LANES)]
        idxb[1, pl.ds(i * LANES, LANES)] = iv + N_USERS_K

    @pl.when(c == 0)
    def _():
        pltpu.async_copy(alo.at[idxb.at[0]], rbu, gsem)
        pltpu.async_copy(alo.at[idxb.at[1]], rbi, gsem)

    @pl.when(c == 1)
    def _():
        pltpu.async_copy(ahi.at[idxb.at[0]], rbu, gsem)
        pltpu.async_copy(ahi.at[idxb.at[1]], rbi, gsem)

    pltpu.make_async_copy(alo.at[idxb.at[0]], rbu, gsem).wait()
    pltpu.make_async_copy(alo.at[idxb.at[1]], rbi, gsem).wait()

    pltpu.sync_copy(rbu, out_u.at[pl.ds(c * BB + base, IDS2)])
    pltpu.sync_copy(rbi, out_i.at[pl.ds(c * BB + base, IDS2)])


_take = pl.kernel(
    _take_body,
    out_type=(jax.ShapeDtypeStruct((2 * BB, 16), jnp.float32),
              jax.ShapeDtypeStruct((2 * BB, 16), jnp.float32)),
    mesh=_mesh,
    scratch_types=[
        pltpu.VMEM((2, IDS2), jnp.int32),
        pltpu.VMEM((IDS2, 16), jnp.float32),
        pltpu.VMEM((IDS2, 16), jnp.float32),
        pltpu.SemaphoreType.DMA,
    ],
    compiler_params=pltpu.CompilerParams(use_tc_tiling_on_sc=False),
    name="lgcn_take_sc",
)


def kernel(user_id, item_id, user_table, item_table, adj_row, adj_col, adj_vals):
    zpad = jnp.zeros((NN_PAD - NN, 16), jnp.float32)
    lo = jnp.concatenate([user_table[:, :16], item_table[:, :16], zpad], axis=0)
    hi = jnp.concatenate([user_table[:, 16:], item_table[:, 16:], zpad], axis=0)
    pad = E_PAD - EE
    rows_p = jnp.concatenate([adj_row, jnp.zeros((pad,), jnp.int32)])
    cols_p = jnp.concatenate([adj_col, jnp.zeros((pad,), jnp.int32)])
    vals_p = jnp.concatenate([adj_vals, jnp.zeros((pad,), jnp.float32)])
    lo_w = jnp.reshape(lo, (NW, 128))
    hi_w = jnp.reshape(hi, (NW, 128))
    alo_w = lo_w
    ahi_w = hi_w
    for layer in range(LAYERS):
        n0, n1 = _spmv(lo, hi, rows_p, cols_p, vals_p)
        blend = _blend_last if layer == LAYERS - 1 else _blend_mid
        lo_w, hi_w, alo_w, ahi_w = blend(
            lo_w, hi_w,
            jnp.reshape(n0, (NW, 128)), jnp.reshape(n1, (NW, 128)),
            alo_w, ahi_w)
        lo = jnp.reshape(lo_w, (NN_PAD, 16))
        hi = jnp.reshape(hi_w, (NN_PAD, 16))
    alo = jnp.reshape(alo_w, (NN_PAD, 16))
    ahi = jnp.reshape(ahi_w, (NN_PAD, 16))
    u2, i2 = _take(alo, ahi, user_id, item_id)
    u_embed = jnp.concatenate([u2[:BB], u2[BB:]], axis=1)
    i_embed = jnp.concatenate([i2[:BB], i2[BB:]], axis=1)
    return (u_embed, i_embed)
